# baseline (speedup 1.0000x reference)
import functools
import math

import jax
import jax.numpy as jnp
from jax import lax
from jax.experimental import pallas as pl
from jax.experimental.pallas import tpu as pltpu
from jax.experimental.pallas import tpu_sc as plsc

_NUM_WORKERS = 32
_CHUNK = 320
_LANES = 16


def _emb_lookup(flat_idx, table4, *, n, d, chunks_per_worker, scale):
    mesh = plsc.VectorSubcoreMesh(core_axis_name="c", subcore_axis_name="s")
    per_worker = n // _NUM_WORKERS
    nc = chunks_per_worker
    assert nc % 2 == 0 and nc >= 4
    npairs = nc // 2
    d4 = 4 * d

    @functools.partial(
        pl.kernel,
        mesh=mesh,
        out_type=jax.ShapeDtypeStruct((n // 4, d4), jnp.float32),
        scratch_types=[
            [pltpu.VMEM((_CHUNK,), jnp.int32) for _ in range(2)],
            [pltpu.VMEM((_CHUNK,), jnp.int32) for _ in range(2)],
            [pltpu.VMEM((_CHUNK, d4), jnp.float32) for _ in range(2)],
            [pltpu.VMEM((_CHUNK // 4, d4), jnp.float32) for _ in range(2)],
            [pltpu.SemaphoreType.DMA for _ in range(2)],
            [pltpu.SemaphoreType.DMA for _ in range(2)],
        ],
        compiler_params=pltpu.CompilerParams(
            use_tc_tiling_on_sc=False, needs_layout_passes=False),
    )
    def body(idx_hbm, table_hbm, out_hbm, idx_v, qv_v, g_v, o_v, gsem, wsem):
        wid = lax.axis_index("s") * 2 + lax.axis_index("c")
        base = wid * per_worker

        def issue(ci, b):
            off = pl.multiple_of(base + ci * _CHUNK, 8)
            pltpu.sync_copy(idx_hbm.at[pl.ds(off, _CHUNK)], idx_v[b])

            def mk_q(g, carry, ib=idx_v[b], qb=qv_v[b]):
                sl = pl.ds(g * _LANES, _LANES)
                qb[sl] = lax.shift_right_logical(ib[sl], 2)
                return carry

            lax.fori_loop(0, _CHUNK // _LANES, mk_q, 0)
            pltpu.async_copy(table_hbm.at[qv_v[b]], g_v[b], gsem[b])

        def wait_gather(b):
            pltpu.make_async_copy(table_hbm.at[qv_v[b]], g_v[b], gsem[b]).wait()

        def wait_writeback(b):
            pltpu.make_async_copy(
                o_v[b], out_hbm.at[pl.ds(0, _CHUNK // 4)], wsem[b]).wait()

        issue(0, 0)
        issue(1, 1)

        @pl.loop(0, npairs)
        def pair_loop(p):
            for b in range(2):
                ci = p * 2 + b
                wait_gather(b)

                @pl.when(p > 0)
                def _():
                    wait_writeback(b)

                def extract(g16, carry, ib=idx_v[b], gb=g_v[b], ob=o_v[b]):
                    rbase = g16 * _LANES
                    rows = lax.iota(jnp.int32, _LANES) + rbase
                    toks = ib[pl.ds(rbase, _LANES)]
                    ph = (toks & 3) * d
                    orow = lax.shift_right_logical(rows, 2)
                    ocol = (rows & 3) * d
                    for j in range(d):
                        vals = plsc.load_gather(gb, [rows, ph + j])
                        plsc.store_scatter(ob, [orow, ocol + j], vals * scale)
                    return carry

                lax.fori_loop(0, _CHUNK // _LANES, extract, 0)
                off4 = pl.multiple_of((base + ci * _CHUNK) // 4, 8)
                pltpu.async_copy(
                    o_v[b], out_hbm.at[pl.ds(off4, _CHUNK // 4)], wsem[b])

                @pl.when(ci + 2 < nc)
                def _():
                    issue(ci + 2, b)

        for b in range(2):
            wait_writeback(b)

    return body(flat_idx, table4)


def kernel(tokens, table):
    b, l = tokens.shape
    v, d = table.shape
    n = b * l
    per_worker = n // _NUM_WORKERS
    flat_idx = tokens.reshape(n).astype(jnp.int32)
    table4 = table.reshape(v // 4, 4 * d)
    out = _emb_lookup(
        flat_idx,
        table4,
        n=n,
        d=d,
        chunks_per_worker=per_worker // _CHUNK,
        scale=math.sqrt(d),
    )
    return out.reshape(b, l, d)


# speedup vs baseline: 1.5311x; 1.5311x over previous
import functools
import math

import jax
import jax.numpy as jnp
from jax import lax
from jax.experimental import pallas as pl
from jax.experimental.pallas import tpu as pltpu
from jax.experimental.pallas import tpu_sc as plsc

_NUM_WORKERS = 32
_CHUNK = 512
_LANES = 16


def _emb_lookup(flat_idx, table, *, b_dim, l_dim, d, scale):
    mesh = plsc.VectorSubcoreMesh(core_axis_name="c", subcore_axis_name="s")
    n = b_dim * l_dim
    per_worker = n // _NUM_WORKERS
    nc = per_worker // _CHUNK
    assert nc % 2 == 0 and nc >= 4
    assert b_dim % _CHUNK == 0 and (b_dim & (b_dim - 1)) == 0
    b_shift = b_dim.bit_length() - 1

    @functools.partial(
        pl.kernel,
        mesh=mesh,
        out_type=jax.ShapeDtypeStruct((l_dim, d, b_dim), jnp.float32),
        scratch_types=[
            pltpu.VMEM((per_worker,), jnp.int32),
            [pltpu.VMEM((_CHUNK, d), jnp.float32) for _ in range(2)],
            [pltpu.VMEM((d, _CHUNK), jnp.float32) for _ in range(2)],
            [pltpu.SemaphoreType.DMA for _ in range(2)],
            [pltpu.SemaphoreType.DMA for _ in range(2)],
        ],
        compiler_params=pltpu.CompilerParams(
            use_tc_tiling_on_sc=False, needs_layout_passes=False),
    )
    def body(idx_hbm, table_hbm, out_hbm, idx_v, rows_v, t_v, gsem, wsem):
        wid = lax.axis_index("s") * 2 + lax.axis_index("c")
        base = wid * per_worker
        pltpu.sync_copy(idx_hbm.at[pl.ds(base, per_worker)], idx_v)

        def start_gather(ci, b):
            off = pl.multiple_of(ci * _CHUNK, 8)
            pltpu.async_copy(
                table_hbm.at[idx_v.at[pl.ds(off, _CHUNK)]], rows_v[b], gsem[b])

        def wait_gather(b):
            pltpu.make_async_copy(
                table_hbm.at[idx_v.at[pl.ds(0, _CHUNK)]], rows_v[b],
                gsem[b]).wait()

        def wait_writeback(b):
            pltpu.make_async_copy(
                t_v[b], out_hbm.at[0, :, pl.ds(0, _CHUNK)], wsem[b]).wait()

        start_gather(0, 0)
        start_gather(1, 1)

        @pl.loop(0, nc // 2)
        def pair_loop(p):
            for b in range(2):
                ci = p * 2 + b
                wait_gather(b)

                @pl.when(p > 0)
                def _():
                    wait_writeback(b)

                def transpose16(g, carry, rows=rows_v[b], t=t_v[b]):
                    r16 = lax.iota(jnp.int32, _LANES) + g * _LANES
                    for j in range(d):
                        vals = plsc.load_gather(
                            rows, [r16, jnp.full((_LANES,), j, jnp.int32)])
                        t[j, pl.ds(g * _LANES, _LANES)] = vals * scale
                    return carry

                lax.fori_loop(0, _CHUNK // _LANES, transpose16, 0)
                gpos = base + ci * _CHUNK
                l_i = lax.shift_right_logical(gpos, b_shift)
                b0 = pl.multiple_of(gpos & (b_dim - 1), _CHUNK)
                pltpu.async_copy(
                    t_v[b], out_hbm.at[l_i, :, pl.ds(b0, _CHUNK)], wsem[b])

                @pl.when(ci + 2 < nc)
                def _():
                    start_gather(ci + 2, b)

        for b in range(2):
            wait_writeback(b)

    return body(flat_idx, table)


def kernel(tokens, table):
    b_dim, l_dim = tokens.shape
    v, d = table.shape
    flat_idx = tokens.T.reshape(b_dim * l_dim).astype(jnp.int32)
    out_nat = _emb_lookup(
        flat_idx,
        table,
        b_dim=b_dim,
        l_dim=l_dim,
        d=d,
        scale=math.sqrt(d),
    )
    return jnp.transpose(out_nat, (2, 0, 1))


# speedup vs baseline: 2.2776x; 1.4876x over previous
import functools
import math

import jax
import jax.numpy as jnp
from jax import lax
from jax.experimental import pallas as pl
from jax.experimental.pallas import tpu as pltpu
from jax.experimental.pallas import tpu_sc as plsc

_NUM_WORKERS = 32
_CHUNK = 512
_LANES = 16


def _emb_lookup(flat_idx, table, *, b_dim, l_dim, d, scale):
    mesh = plsc.VectorSubcoreMesh(core_axis_name="c", subcore_axis_name="s")
    n = b_dim * l_dim
    per_worker = n // _NUM_WORKERS
    nc = per_worker // _CHUNK
    assert nc % 2 == 0 and nc >= 4
    assert b_dim % _CHUNK == 0 and (b_dim & (b_dim - 1)) == 0
    b_shift = b_dim.bit_length() - 1

    @functools.partial(
        pl.kernel,
        mesh=mesh,
        out_type=jax.ShapeDtypeStruct((l_dim, d, b_dim), jnp.float32),
        scratch_types=[
            pltpu.VMEM((per_worker,), jnp.int32),
            [pltpu.VMEM((_CHUNK, d), jnp.float32) for _ in range(2)],
            [pltpu.VMEM((d, _CHUNK + 1), jnp.float32) for _ in range(2)],
            [pltpu.SemaphoreType.DMA for _ in range(2)],
            [pltpu.SemaphoreType.DMA for _ in range(2)],
        ],
        compiler_params=pltpu.CompilerParams(
            use_tc_tiling_on_sc=False, needs_layout_passes=False),
    )
    def body(idx_hbm, table_hbm, out_hbm, idx_v, rows_v, t_v, gsem, wsem):
        wid = lax.axis_index("s") * 2 + lax.axis_index("c")
        base = wid * per_worker
        pltpu.sync_copy(idx_hbm.at[pl.ds(base, per_worker)], idx_v)

        def start_gather(ci, b):
            off = pl.multiple_of(ci * _CHUNK, 8)
            pltpu.async_copy(
                table_hbm.at[idx_v.at[pl.ds(off, _CHUNK)]], rows_v[b], gsem[b])

        def wait_gather(b):
            pltpu.make_async_copy(
                table_hbm.at[idx_v.at[pl.ds(0, _CHUNK)]], rows_v[b],
                gsem[b]).wait()

        def wait_writeback(b):
            pltpu.make_async_copy(
                t_v[b].at[:, pl.ds(0, _CHUNK)],
                out_hbm.at[0, :, pl.ds(0, _CHUNK)], wsem[b]).wait()

        start_gather(0, 0)
        start_gather(1, 1)

        @pl.loop(0, nc // 2)
        def pair_loop(p):
            for b in range(2):
                ci = p * 2 + b
                wait_gather(b)

                @pl.when(p > 0)
                def _():
                    wait_writeback(b)

                jlo = lax.iota(jnp.int32, _LANES)
                jhi = jlo + _LANES

                def transpose4(r4, carry, rows=rows_v[b], t=t_v[b]):
                    for k in range(4):
                        r = r4 * 4 + k
                        rvec = jnp.full((_LANES,), r, jnp.int32)
                        plsc.store_scatter(
                            t, [jlo, rvec], rows[r, pl.ds(0, _LANES)] * scale)
                        plsc.store_scatter(
                            t, [jhi, rvec],
                            rows[r, pl.ds(_LANES, _LANES)] * scale)
                    return carry

                lax.fori_loop(0, _CHUNK // 4, transpose4, 0)
                gpos = base + ci * _CHUNK
                l_i = lax.shift_right_logical(gpos, b_shift)
                b0 = pl.multiple_of(gpos & (b_dim - 1), _CHUNK)
                pltpu.async_copy(
                    t_v[b].at[:, pl.ds(0, _CHUNK)],
                    out_hbm.at[l_i, :, pl.ds(b0, _CHUNK)], wsem[b])

                @pl.when(ci + 2 < nc)
                def _():
                    start_gather(ci + 2, b)

        for b in range(2):
            wait_writeback(b)

    return body(flat_idx, table)


def kernel(tokens, table):
    b_dim, l_dim = tokens.shape
    v, d = table.shape
    flat_idx = tokens.T.reshape(b_dim * l_dim).astype(jnp.int32)
    out_nat = _emb_lookup(
        flat_idx,
        table,
        b_dim=b_dim,
        l_dim=l_dim,
        d=d,
        scale=math.sqrt(d),
    )
    return jnp.transpose(out_nat, (2, 0, 1))
